# Initial kernel scaffold; baseline (speedup 1.0000x reference)
import jax
import jax.numpy as jnp
from jax.experimental import pallas as pl


def kernel(x, edge_index, batch, W1, b1, W2, b2, Wfc, bfc):
    raise NotImplementedError("write your pallas kernel here")


# speedup vs baseline: 8.4453x; 8.4453x over previous
import functools

import jax
import jax.numpy as jnp
from jax import lax
from jax.experimental import pallas as pl
from jax.experimental.pallas import tpu as pltpu
from jax.experimental.pallas import tpu_sc as plsc

NN = 10000
NE = 320000
D = 128
NG = 64
NC = 2
NS = 16
L = 16
NW = NC * NS
K = 128
NCHUNK = 80
EPW = NCHUNK * K
TRASH = NN
NROWS = 10240
DROWS = NROWS // D
BR = 2000


def _sc_mesh():
    return plsc.VectorSubcoreMesh(
        core_axis_name="c", subcore_axis_name="s", num_cores=NC, num_subcores=NS
    )


def _sc_degree(dstp, z1d, ones1d):
    @functools.partial(
        pl.kernel,
        out_type=jax.ShapeDtypeStruct((NC, NROWS), jnp.float32),
        mesh=_sc_mesh(),
        scratch_types=[
            pltpu.VMEM((NCHUNK, K), jnp.int32),
            pltpu.VMEM((K,), jnp.float32),
            pltpu.VMEM_SHARED((NROWS,), jnp.float32),
        ],
    )
    def run(dstp_h, z1d_h, ones_h, out_h, dst_v, ones_v, deg_s):
        c = lax.axis_index("c")
        s = lax.axis_index("s")
        wid = s * NC + c
        pltpu.sync_copy(dstp_h.at[wid], dst_v)
        pltpu.sync_copy(ones_h, ones_v)
        rpt = NROWS // NS
        pltpu.sync_copy(z1d_h.at[pl.ds(s * rpt, rpt)], deg_s.at[pl.ds(s * rpt, rpt)])
        plsc.subcore_barrier()

        def body(g, carry):
            pltpu.sync_copy(ones_v, deg_s.at[dst_v.at[g]], add=True)
            return carry

        lax.fori_loop(0, NCHUNK, body, 0)
        plsc.subcore_barrier()
        pltpu.sync_copy(deg_s.at[pl.ds(s * rpt, rpt)], out_h.at[c, pl.ds(s * rpt, rpt)])

    return run(dstp, z1d, ones1d)


def _sc_scatter(table, srcp, dstp, zrows):
    @functools.partial(
        pl.kernel,
        out_type=jax.ShapeDtypeStruct((NC, NROWS, D), jnp.float32),
        mesh=_sc_mesh(),
        scratch_types=[
            pltpu.VMEM((NCHUNK, K), jnp.int32),
            pltpu.VMEM((NCHUNK, K), jnp.int32),
            pltpu.VMEM((K, D), jnp.float32),
            pltpu.VMEM_SHARED((NROWS, D), jnp.float32),
            pltpu.SemaphoreType.DMA,
        ],
    )
    def run(table_h, srcp_h, dstp_h, zrows_h, acc_h, src_v, dst_v, rows_v, acc_s, sem):
        c = lax.axis_index("c")
        s = lax.axis_index("s")
        wid = s * NC + c
        pltpu.sync_copy(srcp_h.at[wid], src_v)
        pltpu.sync_copy(dstp_h.at[wid], dst_v)
        rpt = NROWS // NS
        pltpu.sync_copy(zrows_h.at[pl.ds(s * rpt, rpt)], acc_s.at[pl.ds(s * rpt, rpt)])
        plsc.subcore_barrier()

        def body(g, carry):
            pltpu.async_copy(table_h.at[src_v.at[g]], rows_v, sem).wait()
            pltpu.sync_copy(rows_v, acc_s.at[dst_v.at[g]], add=True)
            return carry

        lax.fori_loop(0, NCHUNK, body, 0)
        plsc.subcore_barrier()
        pltpu.sync_copy(acc_s.at[pl.ds(s * rpt, rpt)], acc_h.at[c, pl.ds(s * rpt, rpt)])

    return run(table, srcp, dstp, zrows)


def _dinv_of(deg_ref):
    deg = jnp.sum(deg_ref[...], axis=0) + 1.0
    y = lax.rsqrt(deg)
    return y * (1.5 - 0.5 * deg * y * y)


def _bf16_dot(a, b):
    return jnp.dot(a.astype(jnp.bfloat16), b.astype(jnp.bfloat16),
                   preferred_element_type=jnp.float32)


def _mm_scale_body(x_ref, w_ref, deg_ref, o_ref):
    dinv = _dinv_of(deg_ref)
    o_ref[...] = dinv * _bf16_dot(x_ref[...], w_ref[...])


def _tc_mm_scale(x, W, degp):
    return pl.pallas_call(
        _mm_scale_body,
        grid=(NN // BR,),
        in_specs=[
            pl.BlockSpec((BR, D), lambda j: (j, 0)),
            pl.BlockSpec((D, D), lambda j: (0, 0)),
            pl.BlockSpec((NC, BR, 1), lambda j: (0, j, 0)),
        ],
        out_specs=pl.BlockSpec((BR, D), lambda j: (j, 0)),
        out_shape=jax.ShapeDtypeStruct((NN, D), jnp.float32),
    )(x, W, degp)


def _combine_body(hp_ref, acc_ref, deg_ref, b_ref, w_ref, o_ref):
    dinv = _dinv_of(deg_ref)
    z = jnp.maximum(dinv * (hp_ref[...] + acc_ref[0] + acc_ref[1]) + b_ref[...], 0.0)
    o_ref[...] = dinv * _bf16_dot(z, w_ref[...])


def _tc_combine_mm(hp, acc, degp, b, W):
    return pl.pallas_call(
        _combine_body,
        grid=(NN // BR,),
        in_specs=[
            pl.BlockSpec((BR, D), lambda j: (j, 0)),
            pl.BlockSpec((NC, BR, D), lambda j: (0, j, 0)),
            pl.BlockSpec((NC, BR, 1), lambda j: (0, j, 0)),
            pl.BlockSpec((1, D), lambda j: (0, 0)),
            pl.BlockSpec((D, D), lambda j: (0, 0)),
        ],
        out_specs=pl.BlockSpec((BR, D), lambda j: (j, 0)),
        out_shape=jax.ShapeDtypeStruct((NN, D), jnp.float32),
    )(hp, acc, degp, b, W)


def _final_body(hp_ref, acc_ref, deg_ref, b_ref, wfc_ref, bfc_ref, batch_ref,
                o_ref, sums, cnts):
    j = pl.program_id(0)
    dinv = _dinv_of(deg_ref)
    z = jnp.maximum(dinv * (hp_ref[...] + acc_ref[0] + acc_ref[1]) + b_ref[...], 0.0)
    m = (batch_ref[...] == lax.broadcasted_iota(jnp.int32, (1, NG), 1)).astype(jnp.float32)
    ps = lax.dot_general(m, z, (((0,), (0,)), ((), ())),
                         preferred_element_type=jnp.float32,
                         precision=lax.Precision.HIGHEST)
    pc = lax.dot_general(m, jnp.ones((BR, 1), jnp.float32), (((0,), (0,)), ((), ())),
                         preferred_element_type=jnp.float32,
                         precision=lax.Precision.HIGHEST)

    @pl.when(j == 0)
    def _():
        sums[...] = jnp.zeros_like(sums)
        cnts[...] = jnp.zeros_like(cnts)

    sums[...] += ps
    cnts[...] += pc

    @pl.when(j == pl.num_programs(0) - 1)
    def _():
        g = sums[...] / jnp.maximum(cnts[...], 1.0)
        o_ref[...] = _bf16_dot(g, wfc_ref[...]) + bfc_ref[...]


def _tc_final(hp, acc, degp, b, wfc, bfc2, batch2):
    return pl.pallas_call(
        _final_body,
        grid=(NN // BR,),
        in_specs=[
            pl.BlockSpec((BR, D), lambda j: (j, 0)),
            pl.BlockSpec((NC, BR, D), lambda j: (0, j, 0)),
            pl.BlockSpec((NC, BR, 1), lambda j: (0, j, 0)),
            pl.BlockSpec((1, D), lambda j: (0, 0)),
            pl.BlockSpec((D, 1), lambda j: (0, 0)),
            pl.BlockSpec((1, 1), lambda j: (0, 0)),
            pl.BlockSpec((BR, 1), lambda j: (j, 0)),
        ],
        out_specs=pl.BlockSpec((NG, 1), lambda j: (0, 0)),
        out_shape=jax.ShapeDtypeStruct((NG, 1), jnp.float32),
        scratch_shapes=[
            pltpu.VMEM((NG, D), jnp.float32),
            pltpu.VMEM((NG, 1), jnp.float32),
        ],
    )(hp, acc, degp, b, wfc, bfc2, batch2)


def kernel(x, edge_index, batch, W1, b1, W2, b2, Wfc, bfc):
    ei = edge_index.astype(jnp.int32)
    pad = NW * EPW - NE
    src = jnp.concatenate([ei[0], jnp.zeros((pad,), jnp.int32)])
    dst = jnp.concatenate([ei[1], jnp.full((pad,), TRASH, jnp.int32)])
    srcp = src.reshape(NW, NCHUNK, K)
    dstp = dst.reshape(NW, NCHUNK, K)
    dst_flat = dst.reshape(NW, EPW)

    zrows = jnp.zeros((NROWS, D), jnp.float32)
    z1d = jnp.zeros((NROWS,), jnp.float32)

    degp = _sc_degree(dstp, z1d, jnp.ones((K,), jnp.float32))
    degp = degp.reshape(NC, NROWS, 1)

    h1p = _tc_mm_scale(x, W1, degp)
    acc1 = _sc_scatter(h1p, srcp, dstp, zrows)
    h2p = _tc_combine_mm(h1p, acc1, degp, b1.reshape(1, D), W2)
    acc2 = _sc_scatter(h2p, srcp, dstp, zrows)
    out = _tc_final(
        h2p, acc2, degp, b2.reshape(1, D), Wfc,
        bfc.reshape(1, 1), batch.astype(jnp.int32).reshape(NN, 1),
    )
    return out.reshape(NG)


# speedup vs baseline: 9.9567x; 1.1790x over previous
import functools

import jax
import jax.numpy as jnp
from jax import lax
from jax.experimental import pallas as pl
from jax.experimental.pallas import tpu as pltpu
from jax.experimental.pallas import tpu_sc as plsc

NN = 10000
NE = 320000
D = 128
NG = 64
NC = 2
NS = 16
L = 16
NW = NC * NS
K = 128
NCHUNK = 80
EPW = NCHUNK * K
TRASH = NN
NROWS = 10240
DROWS = NROWS // D
BR = 2000


def _sc_mesh():
    return plsc.VectorSubcoreMesh(
        core_axis_name="c", subcore_axis_name="s", num_cores=NC, num_subcores=NS
    )


def _sc_degree(dstp, z1d, ones1d):
    @functools.partial(
        pl.kernel,
        out_type=jax.ShapeDtypeStruct((NC, NROWS), jnp.float32),
        mesh=_sc_mesh(),
        scratch_types=[
            pltpu.VMEM((NCHUNK, K), jnp.int32),
            pltpu.VMEM((K,), jnp.float32),
            pltpu.VMEM_SHARED((NROWS,), jnp.float32),
        ],
    )
    def run(dstp_h, z1d_h, ones_h, out_h, dst_v, ones_v, deg_s):
        c = lax.axis_index("c")
        s = lax.axis_index("s")
        wid = s * NC + c
        pltpu.sync_copy(dstp_h.at[wid], dst_v)
        pltpu.sync_copy(ones_h, ones_v)
        rpt = NROWS // NS
        pltpu.sync_copy(z1d_h.at[pl.ds(s * rpt, rpt)], deg_s.at[pl.ds(s * rpt, rpt)])
        plsc.subcore_barrier()

        def body(g, carry):
            pltpu.sync_copy(ones_v, deg_s.at[dst_v.at[g]], add=True)
            return carry

        lax.fori_loop(0, NCHUNK, body, 0)
        plsc.subcore_barrier()
        pltpu.sync_copy(deg_s.at[pl.ds(s * rpt, rpt)], out_h.at[c, pl.ds(s * rpt, rpt)])

    return run(dstp, z1d, ones1d)


def _sc_scatter(table, srcp, dstp, zrows):
    @functools.partial(
        pl.kernel,
        out_type=jax.ShapeDtypeStruct((NC, NROWS, D), jnp.float32),
        mesh=_sc_mesh(),
        scratch_types=[
            pltpu.VMEM((NCHUNK, K), jnp.int32),
            pltpu.VMEM((NCHUNK, K), jnp.int32),
            pltpu.VMEM((K, D), jnp.float32),
            pltpu.VMEM_SHARED((NROWS, D), jnp.float32),
            pltpu.SemaphoreType.DMA,
        ],
    )
    def run(table_h, srcp_h, dstp_h, zrows_h, acc_h, src_v, dst_v, rows_v, acc_s, sem):
        c = lax.axis_index("c")
        s = lax.axis_index("s")
        wid = s * NC + c
        pltpu.sync_copy(srcp_h.at[wid], src_v)
        pltpu.sync_copy(dstp_h.at[wid], dst_v)
        rpt = NROWS // NS
        pltpu.sync_copy(zrows_h.at[pl.ds(s * rpt, rpt)], acc_s.at[pl.ds(s * rpt, rpt)])
        plsc.subcore_barrier()

        def body(g, carry):
            pltpu.async_copy(table_h.at[src_v.at[g]], rows_v, sem).wait()
            pltpu.sync_copy(rows_v, acc_s.at[dst_v.at[g]], add=True)
            return carry

        lax.fori_loop(0, NCHUNK, body, 0)
        plsc.subcore_barrier()
        pltpu.sync_copy(acc_s.at[pl.ds(s * rpt, rpt)], acc_h.at[c, pl.ds(s * rpt, rpt)])

    return run(table, srcp, dstp, zrows)


def _dinv_of(deg_ref):
    deg = jnp.sum(deg_ref[...], axis=0) + 1.0
    y = lax.rsqrt(deg)
    return y * (1.5 - 0.5 * deg * y * y)


def _bf16_dot(a, b):
    return jnp.dot(a.astype(jnp.bfloat16), b.astype(jnp.bfloat16),
                   preferred_element_type=jnp.float32)


def _mm_scale_body(x_ref, w_ref, deg_ref, o_ref):
    dinv = _dinv_of(deg_ref)
    o_ref[...] = dinv * _bf16_dot(x_ref[...], w_ref[...])


def _tc_mm_scale(x, W, degp):
    return pl.pallas_call(
        _mm_scale_body,
        grid=(NN // BR,),
        in_specs=[
            pl.BlockSpec((BR, D), lambda j: (j, 0)),
            pl.BlockSpec((D, D), lambda j: (0, 0)),
            pl.BlockSpec((NC, BR, 1), lambda j: (0, j, 0)),
        ],
        out_specs=pl.BlockSpec((BR, D), lambda j: (j, 0)),
        out_shape=jax.ShapeDtypeStruct((NN, D), jnp.float32),
    )(x, W, degp)


def _combine_body(hp_ref, acc_ref, deg_ref, b_ref, w_ref, o_ref):
    dinv = _dinv_of(deg_ref)
    z = jnp.maximum(dinv * (hp_ref[...] + acc_ref[0] + acc_ref[1]) + b_ref[...], 0.0)
    o_ref[...] = dinv * _bf16_dot(z, w_ref[...])


def _tc_combine_mm(hp, acc, degp, b, W):
    return pl.pallas_call(
        _combine_body,
        grid=(NN // BR,),
        in_specs=[
            pl.BlockSpec((BR, D), lambda j: (j, 0)),
            pl.BlockSpec((NC, BR, D), lambda j: (0, j, 0)),
            pl.BlockSpec((NC, BR, 1), lambda j: (0, j, 0)),
            pl.BlockSpec((1, D), lambda j: (0, 0)),
            pl.BlockSpec((D, D), lambda j: (0, 0)),
        ],
        out_specs=pl.BlockSpec((BR, D), lambda j: (j, 0)),
        out_shape=jax.ShapeDtypeStruct((NN, D), jnp.float32),
    )(hp, acc, degp, b, W)


def _final_body(hp_ref, acc_ref, deg_ref, b_ref, wfc_ref, bfc_ref, batch_ref,
                o_ref, sums, cnts):
    j = pl.program_id(0)
    dinv = _dinv_of(deg_ref)
    z = jnp.maximum(dinv * (hp_ref[...] + acc_ref[0] + acc_ref[1]) + b_ref[...], 0.0)
    m = (batch_ref[...] == lax.broadcasted_iota(jnp.int32, (1, NG), 1)).astype(jnp.float32)
    ps = lax.dot_general(m, z, (((0,), (0,)), ((), ())),
                         preferred_element_type=jnp.float32,
                         precision=lax.Precision.HIGHEST)
    pc = lax.dot_general(m, jnp.ones((BR, 1), jnp.float32), (((0,), (0,)), ((), ())),
                         preferred_element_type=jnp.float32,
                         precision=lax.Precision.HIGHEST)

    @pl.when(j == 0)
    def _():
        sums[...] = jnp.zeros_like(sums)
        cnts[...] = jnp.zeros_like(cnts)

    sums[...] += ps
    cnts[...] += pc

    @pl.when(j == pl.num_programs(0) - 1)
    def _():
        g = sums[...] / jnp.maximum(cnts[...], 1.0)
        o_ref[...] = _bf16_dot(g, wfc_ref[...]) + bfc_ref[...]


def _tc_final(hp, acc, degp, b, wfc, bfc2, batch2):
    return pl.pallas_call(
        _final_body,
        grid=(NN // BR,),
        in_specs=[
            pl.BlockSpec((BR, D), lambda j: (j, 0)),
            pl.BlockSpec((NC, BR, D), lambda j: (0, j, 0)),
            pl.BlockSpec((NC, BR, 1), lambda j: (0, j, 0)),
            pl.BlockSpec((1, D), lambda j: (0, 0)),
            pl.BlockSpec((D, 1), lambda j: (0, 0)),
            pl.BlockSpec((1, 1), lambda j: (0, 0)),
            pl.BlockSpec((BR, 1), lambda j: (j, 0)),
        ],
        out_specs=pl.BlockSpec((NG, 1), lambda j: (0, 0)),
        out_shape=jax.ShapeDtypeStruct((NG, 1), jnp.float32),
        scratch_shapes=[
            pltpu.VMEM((NG, D), jnp.float32),
            pltpu.VMEM((NG, 1), jnp.float32),
        ],
    )(hp, acc, degp, b, wfc, bfc2, batch2)


def kernel(x, edge_index, batch, W1, b1, W2, b2, Wfc, bfc):
    ei = edge_index.astype(jnp.int32)
    epw_real = NE // NW
    pad = EPW - epw_real
    trash = (TRASH + jnp.arange(NW, dtype=jnp.int32))[:, None]
    src = jnp.concatenate(
        [ei[0].reshape(NW, epw_real), jnp.zeros((NW, pad), jnp.int32)], axis=1)
    dst = jnp.concatenate(
        [ei[1].reshape(NW, epw_real), jnp.broadcast_to(trash, (NW, pad))], axis=1)
    srcp = src.reshape(NW, NCHUNK, K)
    dstp = dst.reshape(NW, NCHUNK, K)

    zrows = jnp.zeros((NROWS, D), jnp.float32)
    z1d = jnp.zeros((NROWS,), jnp.float32)

    degp = _sc_degree(dstp, z1d, jnp.ones((K,), jnp.float32))
    degp = degp.reshape(NC, NROWS, 1)

    h1p = _tc_mm_scale(x, W1, degp)
    acc1 = _sc_scatter(h1p, srcp, dstp, zrows)
    h2p = _tc_combine_mm(h1p, acc1, degp, b1.reshape(1, D), W2)
    acc2 = _sc_scatter(h2p, srcp, dstp, zrows)
    out = _tc_final(
        h2p, acc2, degp, b2.reshape(1, D), Wfc,
        bfc.reshape(1, 1), batch.astype(jnp.int32).reshape(NN, 1),
    )
    return out.reshape(NG)


# speedup vs baseline: 10.8156x; 1.0863x over previous
import functools

import jax
import jax.numpy as jnp
from jax import lax
from jax.experimental import pallas as pl
from jax.experimental.pallas import tpu as pltpu
from jax.experimental.pallas import tpu_sc as plsc

NN = 10000
NE = 320000
D = 128
NG = 64
NC = 2
NS = 16
L = 16
NW = NC * NS
K = 128
NCHUNK = 80
EPW = NCHUNK * K
TRASH = NN
NROWS = 10240
DROWS = NROWS // D
BR = 2000
WIN = 8


def _sc_mesh():
    return plsc.VectorSubcoreMesh(
        core_axis_name="c", subcore_axis_name="s", num_cores=NC, num_subcores=NS
    )


def _sc_degree(dstp, z1d, ones1d):
    @functools.partial(
        pl.kernel,
        out_type=jax.ShapeDtypeStruct((NC, NROWS), jnp.float32),
        mesh=_sc_mesh(),
        scratch_types=[
            pltpu.VMEM((NCHUNK, K), jnp.int32),
            pltpu.VMEM((K,), jnp.float32),
            pltpu.VMEM_SHARED((NROWS,), jnp.float32),
        ],
    )
    def run(dstp_h, z1d_h, ones_h, out_h, dst_v, ones_v, deg_s):
        c = lax.axis_index("c")
        s = lax.axis_index("s")
        wid = s * NC + c
        pltpu.sync_copy(dstp_h.at[wid], dst_v)
        pltpu.sync_copy(ones_h, ones_v)
        rpt = NROWS // NS
        pltpu.sync_copy(z1d_h.at[pl.ds(s * rpt, rpt)], deg_s.at[pl.ds(s * rpt, rpt)])
        plsc.subcore_barrier()

        def body(g, carry):
            pltpu.sync_copy(ones_v, deg_s.at[dst_v.at[g]], add=True)
            return carry

        lax.fori_loop(0, NCHUNK, body, 0)
        plsc.subcore_barrier()
        pltpu.sync_copy(deg_s.at[pl.ds(s * rpt, rpt)], out_h.at[c, pl.ds(s * rpt, rpt)])

    return run(dstp, z1d, ones1d)


def _sc_scatter(table, srcp, dstp, zrows):
    @functools.partial(
        pl.kernel,
        out_type=jax.ShapeDtypeStruct((NC, NROWS, D), jnp.float32),
        mesh=_sc_mesh(),
        scratch_types=[
            pltpu.VMEM((2, WIN, K), jnp.int32),
            pltpu.VMEM((2, WIN, K), jnp.int32),
            pltpu.VMEM((2, K, D), jnp.float32),
            pltpu.VMEM_SHARED((NROWS, D), jnp.float32),
            pltpu.SemaphoreType.DMA,
            pltpu.SemaphoreType.DMA,
        ],
    )
    def run(table_h, srcp_h, dstp_h, zrows_h, acc_h, src_v, dst_v, rows_v, acc_s,
            sem0, sem1):
        c = lax.axis_index("c")
        s = lax.axis_index("s")
        wid = s * NC + c
        sems = [sem0, sem1]
        rpt = NROWS // NS
        pltpu.sync_copy(zrows_h.at[pl.ds(s * rpt, rpt)], acc_s.at[pl.ds(s * rpt, rpt)])
        plsc.subcore_barrier()

        pltpu.sync_copy(srcp_h.at[wid, pl.ds(0, WIN)], src_v.at[0])
        pltpu.sync_copy(dstp_h.at[wid, pl.ds(0, WIN)], dst_v.at[0])
        pltpu.async_copy(table_h.at[src_v.at[0, 0]], rows_v.at[0], sems[0])

        def outer(i, carry):
            p = i % 2

            @pl.when(i + 1 < NCHUNK // WIN)
            def _():
                pltpu.sync_copy(srcp_h.at[wid, pl.ds((i + 1) * WIN, WIN)],
                                src_v.at[1 - p])
                pltpu.sync_copy(dstp_h.at[wid, pl.ds((i + 1) * WIN, WIN)],
                                dst_v.at[1 - p])

            for b in range(WIN):
                bb = b & 1
                pltpu.make_async_copy(
                    table_h.at[src_v.at[p, b]], rows_v.at[bb], sems[bb]).wait()
                if b < WIN - 1:
                    pltpu.async_copy(
                        table_h.at[src_v.at[p, b + 1]], rows_v.at[1 - bb],
                        sems[1 - bb])
                else:
                    @pl.when(i + 1 < NCHUNK // WIN)
                    def _():
                        pltpu.async_copy(
                            table_h.at[src_v.at[1 - p, 0]], rows_v.at[1 - bb],
                            sems[1 - bb])

                pltpu.sync_copy(rows_v.at[bb], acc_s.at[dst_v.at[p, b]], add=True)
            return carry

        lax.fori_loop(0, NCHUNK // WIN, outer, 0)
        plsc.subcore_barrier()
        pltpu.sync_copy(acc_s.at[pl.ds(s * rpt, rpt)], acc_h.at[c, pl.ds(s * rpt, rpt)])

    return run(table, srcp, dstp, zrows)


def _dinv_of(deg_ref):
    deg = jnp.sum(deg_ref[...], axis=0) + 1.0
    y = lax.rsqrt(deg)
    return y * (1.5 - 0.5 * deg * y * y)


def _bf16_dot(a, b):
    return jnp.dot(a.astype(jnp.bfloat16), b.astype(jnp.bfloat16),
                   preferred_element_type=jnp.float32)


def _mm_scale_body(x_ref, w_ref, deg_ref, o_ref):
    dinv = _dinv_of(deg_ref)
    o_ref[...] = dinv * _bf16_dot(x_ref[...], w_ref[...])


def _tc_mm_scale(x, W, degp):
    return pl.pallas_call(
        _mm_scale_body,
        grid=(NN // BR,),
        in_specs=[
            pl.BlockSpec((BR, D), lambda j: (j, 0)),
            pl.BlockSpec((D, D), lambda j: (0, 0)),
            pl.BlockSpec((NC, BR, 1), lambda j: (0, j, 0)),
        ],
        out_specs=pl.BlockSpec((BR, D), lambda j: (j, 0)),
        out_shape=jax.ShapeDtypeStruct((NN, D), jnp.float32),
    )(x, W, degp)


def _combine_body(hp_ref, acc_ref, deg_ref, b_ref, w_ref, o_ref):
    dinv = _dinv_of(deg_ref)
    z = jnp.maximum(dinv * (hp_ref[...] + acc_ref[0] + acc_ref[1]) + b_ref[...], 0.0)
    o_ref[...] = dinv * _bf16_dot(z, w_ref[...])


def _tc_combine_mm(hp, acc, degp, b, W):
    return pl.pallas_call(
        _combine_body,
        grid=(NN // BR,),
        in_specs=[
            pl.BlockSpec((BR, D), lambda j: (j, 0)),
            pl.BlockSpec((NC, BR, D), lambda j: (0, j, 0)),
            pl.BlockSpec((NC, BR, 1), lambda j: (0, j, 0)),
            pl.BlockSpec((1, D), lambda j: (0, 0)),
            pl.BlockSpec((D, D), lambda j: (0, 0)),
        ],
        out_specs=pl.BlockSpec((BR, D), lambda j: (j, 0)),
        out_shape=jax.ShapeDtypeStruct((NN, D), jnp.float32),
    )(hp, acc, degp, b, W)


def _final_body(hp_ref, acc_ref, deg_ref, b_ref, wfc_ref, bfc_ref, batch_ref,
                o_ref, sums, cnts):
    j = pl.program_id(0)
    dinv = _dinv_of(deg_ref)
    z = jnp.maximum(dinv * (hp_ref[...] + acc_ref[0] + acc_ref[1]) + b_ref[...], 0.0)
    m = (batch_ref[...] == lax.broadcasted_iota(jnp.int32, (1, NG), 1)).astype(jnp.float32)
    ps = lax.dot_general(m, z, (((0,), (0,)), ((), ())),
                         preferred_element_type=jnp.float32,
                         precision=lax.Precision.HIGHEST)
    pc = lax.dot_general(m, jnp.ones((BR, 1), jnp.float32), (((0,), (0,)), ((), ())),
                         preferred_element_type=jnp.float32,
                         precision=lax.Precision.HIGHEST)

    @pl.when(j == 0)
    def _():
        sums[...] = jnp.zeros_like(sums)
        cnts[...] = jnp.zeros_like(cnts)

    sums[...] += ps
    cnts[...] += pc

    @pl.when(j == pl.num_programs(0) - 1)
    def _():
        g = sums[...] / jnp.maximum(cnts[...], 1.0)
        o_ref[...] = _bf16_dot(g, wfc_ref[...]) + bfc_ref[...]


def _tc_final(hp, acc, degp, b, wfc, bfc2, batch2):
    return pl.pallas_call(
        _final_body,
        grid=(NN // BR,),
        in_specs=[
            pl.BlockSpec((BR, D), lambda j: (j, 0)),
            pl.BlockSpec((NC, BR, D), lambda j: (0, j, 0)),
            pl.BlockSpec((NC, BR, 1), lambda j: (0, j, 0)),
            pl.BlockSpec((1, D), lambda j: (0, 0)),
            pl.BlockSpec((D, 1), lambda j: (0, 0)),
            pl.BlockSpec((1, 1), lambda j: (0, 0)),
            pl.BlockSpec((BR, 1), lambda j: (j, 0)),
        ],
        out_specs=pl.BlockSpec((NG, 1), lambda j: (0, 0)),
        out_shape=jax.ShapeDtypeStruct((NG, 1), jnp.float32),
        scratch_shapes=[
            pltpu.VMEM((NG, D), jnp.float32),
            pltpu.VMEM((NG, 1), jnp.float32),
        ],
    )(hp, acc, degp, b, wfc, bfc2, batch2)


def kernel(x, edge_index, batch, W1, b1, W2, b2, Wfc, bfc):
    ei = edge_index.astype(jnp.int32)
    epw_real = NE // NW
    pad = EPW - epw_real
    trash = (TRASH + jnp.arange(NW, dtype=jnp.int32))[:, None]
    src = jnp.concatenate(
        [ei[0].reshape(NW, epw_real), jnp.zeros((NW, pad), jnp.int32)], axis=1)
    dst = jnp.concatenate(
        [ei[1].reshape(NW, epw_real), jnp.broadcast_to(trash, (NW, pad))], axis=1)
    srcp = src.reshape(NW, NCHUNK, K)
    dstp = dst.reshape(NW, NCHUNK, K)

    zrows = jnp.zeros((NROWS, D), jnp.float32)
    z1d = jnp.zeros((NROWS,), jnp.float32)

    degp = _sc_degree(dstp, z1d, jnp.ones((K,), jnp.float32))
    degp = degp.reshape(NC, NROWS, 1)

    h1p = _tc_mm_scale(x, W1, degp)
    acc1 = _sc_scatter(h1p, srcp, dstp, zrows)
    h2p = _tc_combine_mm(h1p, acc1, degp, b1.reshape(1, D), W2)
    acc2 = _sc_scatter(h2p, srcp, dstp, zrows)
    out = _tc_final(
        h2p, acc2, degp, b2.reshape(1, D), Wfc,
        bfc.reshape(1, 1), batch.astype(jnp.int32).reshape(NN, 1),
    )
    return out.reshape(NG)


# speedup vs baseline: 10.8952x; 1.0074x over previous
import functools

import jax
import jax.numpy as jnp
from jax import lax
from jax.experimental import pallas as pl
from jax.experimental.pallas import tpu as pltpu
from jax.experimental.pallas import tpu_sc as plsc

NN = 10000
NE = 320000
D = 128
NG = 64
NC = 2
NS = 16
L = 16
NW = NC * NS
K = 128
NCHUNK = 80
EPW = NCHUNK * K
TRASH = NN
NROWS = 10240
DROWS = NROWS // D
BR = 2000
WIN = 8


def _sc_mesh():
    return plsc.VectorSubcoreMesh(
        core_axis_name="c", subcore_axis_name="s", num_cores=NC, num_subcores=NS
    )


def _sc_degree(dstp, z1d, ones1d):
    @functools.partial(
        pl.kernel,
        out_type=jax.ShapeDtypeStruct((NC, NROWS), jnp.float32),
        mesh=_sc_mesh(),
        scratch_types=[
            pltpu.VMEM((NCHUNK, K), jnp.int32),
            pltpu.VMEM((K,), jnp.float32),
            pltpu.VMEM_SHARED((NROWS,), jnp.float32),
        ],
    )
    def run(dstp_h, z1d_h, ones_h, out_h, dst_v, ones_v, deg_s):
        c = lax.axis_index("c")
        s = lax.axis_index("s")
        wid = s * NC + c
        pltpu.sync_copy(dstp_h.at[wid], dst_v)
        pltpu.sync_copy(ones_h, ones_v)
        rpt = NROWS // NS
        pltpu.sync_copy(z1d_h.at[pl.ds(s * rpt, rpt)], deg_s.at[pl.ds(s * rpt, rpt)])
        plsc.subcore_barrier()

        def body(g, carry):
            pltpu.sync_copy(ones_v, deg_s.at[dst_v.at[g]], add=True)
            return carry

        lax.fori_loop(0, NCHUNK, body, 0)
        plsc.subcore_barrier()
        pltpu.sync_copy(deg_s.at[pl.ds(s * rpt, rpt)], out_h.at[c, pl.ds(s * rpt, rpt)])

    return run(dstp, z1d, ones1d)


def _sc_scatter(table, srcp, dstp, zrows):
    @functools.partial(
        pl.kernel,
        out_type=jax.ShapeDtypeStruct((NC, NROWS, D), jnp.float32),
        mesh=_sc_mesh(),
        scratch_types=[
            pltpu.VMEM((2, WIN, K), jnp.int32),
            pltpu.VMEM((2, WIN, K), jnp.int32),
            pltpu.VMEM((2, K, D), jnp.float32),
            pltpu.VMEM_SHARED((NROWS, D), jnp.float32),
            pltpu.SemaphoreType.DMA,
            pltpu.SemaphoreType.DMA,
            pltpu.SemaphoreType.DMA,
            pltpu.SemaphoreType.DMA,
        ],
    )
    def run(table_h, srcp_h, dstp_h, zrows_h, acc_h, src_v, dst_v, rows_v, acc_s,
            sem0, sem1, sem2, sem3):
        c = lax.axis_index("c")
        s = lax.axis_index("s")
        wid = s * NC + c
        sems = [sem0, sem1]
        ssems = [sem2, sem3]
        rpt = NROWS // NS
        pltpu.sync_copy(zrows_h.at[pl.ds(s * rpt, rpt)], acc_s.at[pl.ds(s * rpt, rpt)])
        plsc.subcore_barrier()

        pltpu.sync_copy(srcp_h.at[wid, pl.ds(0, WIN)], src_v.at[0])
        pltpu.sync_copy(dstp_h.at[wid, pl.ds(0, WIN)], dst_v.at[0])
        pltpu.async_copy(table_h.at[src_v.at[0, 0]], rows_v.at[0], sems[0])

        def outer(i, carry):
            p = i % 2

            @pl.when(i + 1 < NCHUNK // WIN)
            def _():
                pltpu.sync_copy(srcp_h.at[wid, pl.ds((i + 1) * WIN, WIN)],
                                src_v.at[1 - p])
                pltpu.sync_copy(dstp_h.at[wid, pl.ds((i + 1) * WIN, WIN)],
                                dst_v.at[1 - p])

            for b in range(WIN):
                bb = b & 1
                pltpu.make_async_copy(
                    table_h.at[src_v.at[p, b]], rows_v.at[bb], sems[bb]).wait()
                pltpu.async_copy(rows_v.at[bb], acc_s.at[dst_v.at[p, b]],
                                 ssems[bb], add=True)

                def _drain_scatter(buf):
                    pltpu.make_async_copy(
                        table_h.at[src_v.at[p, b]], rows_v.at[buf],
                        ssems[buf]).wait()

                if b == 0:
                    @pl.when(i > 0)
                    def _():
                        _drain_scatter(1 - bb)
                else:
                    _drain_scatter(1 - bb)

                if b < WIN - 1:
                    pltpu.async_copy(
                        table_h.at[src_v.at[p, b + 1]], rows_v.at[1 - bb],
                        sems[1 - bb])
                else:
                    @pl.when(i + 1 < NCHUNK // WIN)
                    def _():
                        pltpu.async_copy(
                            table_h.at[src_v.at[1 - p, 0]], rows_v.at[1 - bb],
                            sems[1 - bb])

            return carry

        lax.fori_loop(0, NCHUNK // WIN, outer, 0)
        pltpu.make_async_copy(table_h.at[src_v.at[0, 0]], rows_v.at[1],
                              ssems[1]).wait()
        plsc.subcore_barrier()
        pltpu.sync_copy(acc_s.at[pl.ds(s * rpt, rpt)], acc_h.at[c, pl.ds(s * rpt, rpt)])

    return run(table, srcp, dstp, zrows)


def _dinv_of(deg_ref):
    deg = jnp.sum(deg_ref[...], axis=0) + 1.0
    y = lax.rsqrt(deg)
    return y * (1.5 - 0.5 * deg * y * y)


def _bf16_dot(a, b):
    return jnp.dot(a.astype(jnp.bfloat16), b.astype(jnp.bfloat16),
                   preferred_element_type=jnp.float32)


def _mm_scale_body(x_ref, w_ref, deg_ref, o_ref):
    dinv = _dinv_of(deg_ref)
    o_ref[...] = dinv * _bf16_dot(x_ref[...], w_ref[...])


def _tc_mm_scale(x, W, degp):
    return pl.pallas_call(
        _mm_scale_body,
        grid=(NN // BR,),
        in_specs=[
            pl.BlockSpec((BR, D), lambda j: (j, 0)),
            pl.BlockSpec((D, D), lambda j: (0, 0)),
            pl.BlockSpec((NC, BR, 1), lambda j: (0, j, 0)),
        ],
        out_specs=pl.BlockSpec((BR, D), lambda j: (j, 0)),
        out_shape=jax.ShapeDtypeStruct((NN, D), jnp.float32),
    )(x, W, degp)


def _combine_body(hp_ref, acc_ref, deg_ref, b_ref, w_ref, o_ref):
    dinv = _dinv_of(deg_ref)
    z = jnp.maximum(dinv * (hp_ref[...] + acc_ref[0] + acc_ref[1]) + b_ref[...], 0.0)
    o_ref[...] = dinv * _bf16_dot(z, w_ref[...])


def _tc_combine_mm(hp, acc, degp, b, W):
    return pl.pallas_call(
        _combine_body,
        grid=(NN // BR,),
        in_specs=[
            pl.BlockSpec((BR, D), lambda j: (j, 0)),
            pl.BlockSpec((NC, BR, D), lambda j: (0, j, 0)),
            pl.BlockSpec((NC, BR, 1), lambda j: (0, j, 0)),
            pl.BlockSpec((1, D), lambda j: (0, 0)),
            pl.BlockSpec((D, D), lambda j: (0, 0)),
        ],
        out_specs=pl.BlockSpec((BR, D), lambda j: (j, 0)),
        out_shape=jax.ShapeDtypeStruct((NN, D), jnp.float32),
    )(hp, acc, degp, b, W)


def _final_body(hp_ref, acc_ref, deg_ref, b_ref, wfc_ref, bfc_ref, batch_ref,
                o_ref, sums, cnts):
    j = pl.program_id(0)
    dinv = _dinv_of(deg_ref)
    z = jnp.maximum(dinv * (hp_ref[...] + acc_ref[0] + acc_ref[1]) + b_ref[...], 0.0)
    m = (batch_ref[...] == lax.broadcasted_iota(jnp.int32, (1, NG), 1)).astype(jnp.float32)
    ps = lax.dot_general(m, z, (((0,), (0,)), ((), ())),
                         preferred_element_type=jnp.float32,
                         precision=lax.Precision.HIGHEST)
    pc = lax.dot_general(m, jnp.ones((BR, 1), jnp.float32), (((0,), (0,)), ((), ())),
                         preferred_element_type=jnp.float32,
                         precision=lax.Precision.HIGHEST)

    @pl.when(j == 0)
    def _():
        sums[...] = jnp.zeros_like(sums)
        cnts[...] = jnp.zeros_like(cnts)

    sums[...] += ps
    cnts[...] += pc

    @pl.when(j == pl.num_programs(0) - 1)
    def _():
        g = sums[...] / jnp.maximum(cnts[...], 1.0)
        o_ref[...] = _bf16_dot(g, wfc_ref[...]) + bfc_ref[...]


def _tc_final(hp, acc, degp, b, wfc, bfc2, batch2):
    return pl.pallas_call(
        _final_body,
        grid=(NN // BR,),
        in_specs=[
            pl.BlockSpec((BR, D), lambda j: (j, 0)),
            pl.BlockSpec((NC, BR, D), lambda j: (0, j, 0)),
            pl.BlockSpec((NC, BR, 1), lambda j: (0, j, 0)),
            pl.BlockSpec((1, D), lambda j: (0, 0)),
            pl.BlockSpec((D, 1), lambda j: (0, 0)),
            pl.BlockSpec((1, 1), lambda j: (0, 0)),
            pl.BlockSpec((BR, 1), lambda j: (j, 0)),
        ],
        out_specs=pl.BlockSpec((NG, 1), lambda j: (0, 0)),
        out_shape=jax.ShapeDtypeStruct((NG, 1), jnp.float32),
        scratch_shapes=[
            pltpu.VMEM((NG, D), jnp.float32),
            pltpu.VMEM((NG, 1), jnp.float32),
        ],
    )(hp, acc, degp, b, wfc, bfc2, batch2)


def kernel(x, edge_index, batch, W1, b1, W2, b2, Wfc, bfc):
    ei = edge_index.astype(jnp.int32)
    epw_real = NE // NW
    pad = EPW - epw_real
    trash = (TRASH + jnp.arange(NW, dtype=jnp.int32))[:, None]
    src = jnp.concatenate(
        [ei[0].reshape(NW, epw_real), jnp.zeros((NW, pad), jnp.int32)], axis=1)
    dst = jnp.concatenate(
        [ei[1].reshape(NW, epw_real), jnp.broadcast_to(trash, (NW, pad))], axis=1)
    srcp = src.reshape(NW, NCHUNK, K)
    dstp = dst.reshape(NW, NCHUNK, K)

    zrows = jnp.zeros((NROWS, D), jnp.float32)
    z1d = jnp.zeros((NROWS,), jnp.float32)

    degp = _sc_degree(dstp, z1d, jnp.ones((K,), jnp.float32))
    degp = degp.reshape(NC, NROWS, 1)

    h1p = _tc_mm_scale(x, W1, degp)
    acc1 = _sc_scatter(h1p, srcp, dstp, zrows)
    h2p = _tc_combine_mm(h1p, acc1, degp, b1.reshape(1, D), W2)
    acc2 = _sc_scatter(h2p, srcp, dstp, zrows)
    out = _tc_final(
        h2p, acc2, degp, b2.reshape(1, D), Wfc,
        bfc.reshape(1, 1), batch.astype(jnp.int32).reshape(NN, 1),
    )
    return out.reshape(NG)
